# Initial kernel scaffold; baseline (speedup 1.0000x reference)
import jax
import jax.numpy as jnp
from jax.experimental import pallas as pl


def kernel(x, codebooks):
    raise NotImplementedError("write your pallas kernel here")


# speedup vs baseline: 3.9514x; 3.9514x over previous
import functools

import jax
import jax.numpy as jnp
from jax.experimental import pallas as pl
from jax.experimental.pallas import tpu as pltpu

_COMMIT = 0.25


def _vq_block_kernel(x_ref, cb_ref, q_ref, idx_ref, csum_ref, *, n_steps):
    G, M, K, d = cb_ref.shape
    BLK = x_ref.shape[0]
    step = pl.program_id(0)

    @pl.when(step == 0)
    def _init():
        csum_ref[:, :] = jnp.zeros((1, 1), jnp.float32)

    acc = jnp.float32(0.0)
    dn_gather = (((1,), (0,)), ((), ()))
    for g in range(G):
        r = x_ref[:, g * d:(g + 1) * d]
        rec = jnp.zeros_like(r)
        for m in range(M):
            E = cb_ref[g, m]
            r2 = jnp.sum(r * r, axis=1, keepdims=True)
            e2 = jnp.sum(E * E, axis=1)
            dots = jax.lax.dot_general(
                r.astype(jnp.bfloat16), E.astype(jnp.bfloat16),
                (((1,), (1,)), ((), ())),
                preferred_element_type=jnp.float32)
            dists = r2 - 2.0 * dots + e2[None, :]
            idx = jnp.argmin(dists, axis=1).astype(jnp.int32)
            oh = (jax.lax.broadcasted_iota(jnp.int32, (BLK, K), 1)
                  == idx[:, None]).astype(jnp.float32)
            e_hi = E.astype(jnp.bfloat16).astype(jnp.float32)
            e_mid_f = E - e_hi
            e_mid = e_mid_f.astype(jnp.bfloat16).astype(jnp.float32)
            e_lo = e_mid_f - e_mid
            q = (jax.lax.dot_general(oh, e_hi, dn_gather,
                                     preferred_element_type=jnp.float32)
                 + jax.lax.dot_general(oh, e_mid, dn_gather,
                                       preferred_element_type=jnp.float32)
                 + jax.lax.dot_general(oh, e_lo, dn_gather,
                                       preferred_element_type=jnp.float32))
            acc += jnp.sum((q - r) ** 2)
            q_st = r + (q - r)
            rec = rec + q_st
            r = r - q_st
            idx_ref[:, g * M + m:g * M + m + 1] = idx[:, None]
        q_ref[:, g * d:(g + 1) * d] = rec
    csum_ref[:, :] += acc.reshape(1, 1)


def kernel(x, codebooks):
    B, T, D = x.shape
    G, M, K, d = codebooks.shape
    N = B * T
    GM = G * M
    BLK = 512
    xf = x.reshape(N, D)

    grid = (N // BLK,)
    quant, idx, csum = pl.pallas_call(
        functools.partial(_vq_block_kernel, n_steps=N // BLK),
        grid=grid,
        in_specs=[
            pl.BlockSpec((BLK, D), lambda i: (i, 0)),
            pl.BlockSpec((G, M, K, d), lambda i: (0, 0, 0, 0)),
        ],
        out_specs=[
            pl.BlockSpec((BLK, D), lambda i: (i, 0)),
            pl.BlockSpec((BLK, GM), lambda i: (i, 0)),
            pl.BlockSpec((1, 1), lambda i: (0, 0)),
        ],
        out_shape=[
            jax.ShapeDtypeStruct((N, D), jnp.float32),
            jax.ShapeDtypeStruct((N, GM), jnp.int32),
            jax.ShapeDtypeStruct((1, 1), jnp.float32),
        ],
        compiler_params=pltpu.CompilerParams(
            dimension_semantics=("arbitrary",)),
    )(xf, codebooks)

    quantized = quant.reshape(B, T, D)
    indices = idx.reshape(B, T, GM)
    commit = csum[0, 0] * (_COMMIT / (N * d))
    return quantized, indices, commit


# speedup vs baseline: 4.8374x; 1.2242x over previous
import functools

import jax
import jax.numpy as jnp
from jax.experimental import pallas as pl
from jax.experimental.pallas import tpu as pltpu

_COMMIT = 0.25


def _vq_block_kernel(x_ref, cb_ref, q_ref, idx_ref, csum_ref, *, n_steps):
    G, M, K, d = cb_ref.shape
    BLK = x_ref.shape[0]
    step = pl.program_id(0)

    @pl.when(step == 0)
    def _init():
        csum_ref[:, :] = jnp.zeros((1, 1), jnp.float32)

    acc = jnp.float32(0.0)
    dn_gather = (((1,), (0,)), ((), ()))
    for g in range(G):
        r = x_ref[:, g * d:(g + 1) * d]
        rec = jnp.zeros_like(r)
        for m in range(M):
            E = cb_ref[g, m]
            r2 = jnp.sum(r * r, axis=1, keepdims=True)
            e2 = jnp.sum(E * E, axis=1)
            dots = jax.lax.dot_general(
                r.astype(jnp.bfloat16), E.astype(jnp.bfloat16),
                (((1,), (1,)), ((), ())),
                preferred_element_type=jnp.float32)
            dists = r2 - 2.0 * dots + e2[None, :]
            idx = jnp.argmin(dists, axis=1).astype(jnp.int32)
            oh = (jax.lax.broadcasted_iota(jnp.int32, (BLK, K), 1)
                  == idx[:, None]).astype(jnp.float32)
            e_hi = E.astype(jnp.bfloat16).astype(jnp.float32)
            e_mid_f = E - e_hi
            e_mid = e_mid_f.astype(jnp.bfloat16).astype(jnp.float32)
            e_lo = e_mid_f - e_mid
            q = (jax.lax.dot_general(oh, e_hi, dn_gather,
                                     preferred_element_type=jnp.float32)
                 + jax.lax.dot_general(oh, e_mid, dn_gather,
                                       preferred_element_type=jnp.float32)
                 + jax.lax.dot_general(oh, e_lo, dn_gather,
                                       preferred_element_type=jnp.float32))
            acc += jnp.sum((q - r) ** 2)
            q_st = r + (q - r)
            rec = rec + q_st
            r = r - q_st
            idx_ref[:, g * M + m:g * M + m + 1] = idx[:, None]
        q_ref[:, g * d:(g + 1) * d] = rec
    csum_ref[:, :] += acc.reshape(1, 1)


def kernel(x, codebooks):
    B, T, D = x.shape
    G, M, K, d = codebooks.shape
    N = B * T
    GM = G * M
    BLK = 2048
    xf = x.reshape(N, D)

    grid = (N // BLK,)
    quant, idx, csum = pl.pallas_call(
        functools.partial(_vq_block_kernel, n_steps=N // BLK),
        grid=grid,
        in_specs=[
            pl.BlockSpec((BLK, D), lambda i: (i, 0)),
            pl.BlockSpec((G, M, K, d), lambda i: (0, 0, 0, 0)),
        ],
        out_specs=[
            pl.BlockSpec((BLK, D), lambda i: (i, 0)),
            pl.BlockSpec((BLK, GM), lambda i: (i, 0)),
            pl.BlockSpec((1, 1), lambda i: (0, 0)),
        ],
        out_shape=[
            jax.ShapeDtypeStruct((N, D), jnp.float32),
            jax.ShapeDtypeStruct((N, GM), jnp.int32),
            jax.ShapeDtypeStruct((1, 1), jnp.float32),
        ],
        compiler_params=pltpu.CompilerParams(
            dimension_semantics=("arbitrary",)),
    )(xf, codebooks)

    quantized = quant.reshape(B, T, D)
    indices = idx.reshape(B, T, GM)
    commit = csum[0, 0] * (_COMMIT / (N * d))
    return quantized, indices, commit
